# baseline (speedup 1.0000x reference)
import functools

import jax
import jax.numpy as jnp
from jax.experimental import pallas as pl
from jax.experimental.pallas import tpu as pltpu


def _khop_body(a1_ref, a2_ref, x_ref, wk_ref, b_ref, out_ref, *, block_rows, half):
    i = pl.program_id(0)
    rb = pl.program_id(1)

    s1 = jnp.dot(a1_ref[0], x_ref[...], preferred_element_type=jnp.float32)
    c1 = jnp.dot(s1, wk_ref[0], preferred_element_type=jnp.float32)
    s2 = jnp.dot(a2_ref[0], x_ref[...], preferred_element_type=jnp.float32)
    c2 = jnp.dot(s2, wk_ref[0], preferred_element_type=jnp.float32)

    r1 = pl.ds(rb * block_rows, block_rows)
    r2 = pl.ds(half + rb * block_rows, block_rows)

    @pl.when(i == 0)
    def _():
        out_ref[r1, :] = c1 + b_ref[...]
        out_ref[r2, :] = c2 + b_ref[...]

    @pl.when(i > 0)
    def _():
        out_ref[r1, :] += c1
        out_ref[r2, :] += c2


@functools.partial(jax.jit, static_argnames=("block_rows",))
def _khop(x, adj_list, wk, b2, block_rows):
    k, n, _ = adj_list.shape
    d_in = x.shape[1]
    d_out = wk.shape[2]
    half = n // 2
    nrb = half // block_rows
    grid = (k, nrb)
    return pl.pallas_call(
        functools.partial(_khop_body, block_rows=block_rows, half=half),
        grid=grid,
        in_specs=[
            pl.BlockSpec((1, block_rows, n), lambda i, rb: (i, rb, 0)),
            pl.BlockSpec((1, block_rows, n), lambda i, rb, _nrb=nrb: (i, rb + _nrb, 0)),
            pl.BlockSpec((n, d_in), lambda i, rb: (0, 0)),
            pl.BlockSpec((1, d_in, d_out), lambda i, rb: (i, 0, 0)),
            pl.BlockSpec((1, d_out), lambda i, rb: (0, 0)),
        ],
        out_specs=pl.BlockSpec((n, d_out), lambda i, rb: (0, 0)),
        out_shape=jax.ShapeDtypeStruct((n, d_out), jnp.float32),
        compiler_params=pltpu.CompilerParams(
            dimension_semantics=("arbitrary", "arbitrary"),
            vmem_limit_bytes=100 * 1024 * 1024,
        ),
    )(adj_list, adj_list, x, wk, b2)


def kernel(x, adj_list, W, b):
    k, n, _ = adj_list.shape
    d_in = x.shape[1]
    d_out = W.shape[0]
    wk = W.reshape(d_out, k, d_in).transpose(1, 2, 0)
    b2 = b.reshape(1, d_out)
    return _khop(x, adj_list, wk, b2, block_rows=512)


# speedup vs baseline: 1.0652x; 1.0652x over previous
import functools

import jax
import jax.numpy as jnp
from jax.experimental import pallas as pl
from jax.experimental.pallas import tpu as pltpu


def _khop_body(a_ref, x_ref, wk_ref, b_ref, out_ref, *, block_rows):
    i = pl.program_id(0)
    rb = pl.program_id(1)
    s = jnp.dot(a_ref[0], x_ref[...], preferred_element_type=jnp.float32)
    contrib = jnp.dot(s, wk_ref[0], preferred_element_type=jnp.float32)
    rows = pl.ds(rb * block_rows, block_rows)

    @pl.when(i == 0)
    def _():
        out_ref[rows, :] = contrib + b_ref[...]

    @pl.when(i > 0)
    def _():
        out_ref[rows, :] += contrib


@functools.partial(jax.jit, static_argnames=("block_rows",))
def _khop(x, adj_list, wk, b2, block_rows):
    k, n, _ = adj_list.shape
    d_in = x.shape[1]
    d_out = wk.shape[2]
    grid = (k, n // block_rows)
    return pl.pallas_call(
        functools.partial(_khop_body, block_rows=block_rows),
        grid=grid,
        in_specs=[
            pl.BlockSpec((1, block_rows, n), lambda i, rb: (i, rb, 0)),
            pl.BlockSpec((n, d_in), lambda i, rb: (0, 0)),
            pl.BlockSpec((1, d_in, d_out), lambda i, rb: (i, 0, 0)),
            pl.BlockSpec((1, d_out), lambda i, rb: (0, 0)),
        ],
        out_specs=pl.BlockSpec((n, d_out), lambda i, rb: (0, 0)),
        out_shape=jax.ShapeDtypeStruct((n, d_out), jnp.float32),
        compiler_params=pltpu.CompilerParams(
            dimension_semantics=("arbitrary", "arbitrary"),
            vmem_limit_bytes=100 * 1024 * 1024,
        ),
    )(adj_list, x, wk, b2)


def kernel(x, adj_list, W, b):
    k, n, _ = adj_list.shape
    d_in = x.shape[1]
    d_out = W.shape[0]
    wk = W.reshape(d_out, k, d_in).transpose(1, 2, 0)
    b2 = b.reshape(1, d_out)
    return _khop(x, adj_list, wk, b2, block_rows=512)
